# baseline (speedup 1.0000x reference)
import functools

import jax
import jax.numpy as jnp
from jax import lax
from jax.experimental import pallas as pl
from jax.experimental.pallas import tpu as pltpu
from jax.experimental.pallas import tpu_sc as plsc

D_MODEL = 1024
NUM_CORES = 2
NUM_SUBCORES = 16
NUM_WORKERS = NUM_CORES * NUM_SUBCORES
VEC = 16


def kernel(x, pos_embedding, start_pos):
    batch, seq_len, d_model = x.shape
    rows_per_worker = seq_len // NUM_WORKERS
    chunk = min(32, rows_per_worker)
    n_chunks = rows_per_worker // chunk
    chunk_words = chunk * d_model
    n_vecs = chunk_words // VEC

    x_flat = x.reshape(-1)
    pos_flat = pos_embedding.reshape(-1)
    sp = jnp.full((16,), start_pos, dtype=jnp.int32)

    mesh = plsc.VectorSubcoreMesh(
        core_axis_name="c", subcore_axis_name="s",
        num_cores=NUM_CORES, num_subcores=NUM_SUBCORES)

    @functools.partial(
        pl.kernel,
        out_type=jax.ShapeDtypeStruct((batch * seq_len * d_model,),
                                      jnp.float32),
        mesh=mesh,
        scratch_types=[
            pltpu.VMEM((16,), jnp.int32),
            pltpu.VMEM((chunk_words,), jnp.float32),
            pltpu.VMEM((chunk_words,), jnp.float32),
        ],
    )
    def run(x_hbm, pos_hbm, sp_hbm, out_hbm, sp_vmem, posbuf, xbuf):
        cid = lax.axis_index("c")
        sid = lax.axis_index("s")
        wid = sid * NUM_CORES + cid
        pltpu.sync_copy(sp_hbm, sp_vmem)
        s0 = sp_vmem[...][0]
        base = wid * rows_per_worker
        for c in range(n_chunks):
            row0 = base + c * chunk
            pltpu.sync_copy(
                pos_hbm.at[pl.ds((s0 + row0) * d_model, chunk_words)],
                posbuf)
            for b in range(batch):
                off = (b * seq_len + row0) * d_model
                pltpu.sync_copy(x_hbm.at[pl.ds(off, chunk_words)], xbuf)

                @plsc.parallel_loop(0, n_vecs, 1, unroll=8)
                def body(i):
                    o = i * VEC
                    xbuf[pl.ds(o, VEC)] = (
                        xbuf[pl.ds(o, VEC)] + posbuf[pl.ds(o, VEC)])

                pltpu.sync_copy(xbuf, out_hbm.at[pl.ds(off, chunk_words)])

    out = run(x_flat, pos_flat, sp)
    return out.reshape(batch, seq_len, d_model)


# speedup vs baseline: 2.0152x; 2.0152x over previous
import functools

import jax
import jax.numpy as jnp
from jax import lax
from jax.experimental import pallas as pl
from jax.experimental.pallas import tpu as pltpu
from jax.experimental.pallas import tpu_sc as plsc

NUM_CORES = 2
NUM_SUBCORES = 16
NUM_WORKERS = NUM_CORES * NUM_SUBCORES
VEC = 16


def kernel(x, pos_embedding, start_pos):
    batch, seq_len, d_model = x.shape
    rows_per_worker = seq_len // NUM_WORKERS
    chunk = min(32, rows_per_worker)
    n_chunks = rows_per_worker // chunk
    vecs_per_row = d_model // VEC

    sp = jnp.full((16,), start_pos, dtype=jnp.int32)

    mesh = plsc.VectorSubcoreMesh(
        core_axis_name="c", subcore_axis_name="s",
        num_cores=NUM_CORES, num_subcores=NUM_SUBCORES)

    @functools.partial(
        pl.kernel,
        out_type=jax.ShapeDtypeStruct((batch, seq_len, d_model),
                                      jnp.float32),
        mesh=mesh,
        scratch_types=[
            pltpu.VMEM((16,), jnp.int32),
            pltpu.VMEM((chunk, d_model), jnp.float32),
            pltpu.VMEM((chunk, d_model), jnp.float32),
        ],
        compiler_params=pltpu.CompilerParams(use_tc_tiling_on_sc=True),
    )
    def run(x_hbm, pos_hbm, sp_hbm, out_hbm, sp_vmem, posbuf, xbuf):
        cid = lax.axis_index("c")
        sid = lax.axis_index("s")
        wid = sid * NUM_CORES + cid
        pltpu.sync_copy(sp_hbm, sp_vmem)
        s0 = sp_vmem[...][0]
        base = wid * rows_per_worker
        for c in range(n_chunks):
            row0 = pl.multiple_of(base + c * chunk, chunk)
            prow0 = pl.multiple_of(s0 + row0, 8)
            pltpu.sync_copy(pos_hbm.at[pl.ds(prow0, chunk)], posbuf)
            for b in range(batch):
                pltpu.sync_copy(x_hbm.at[b, pl.ds(row0, chunk)], xbuf)

                @plsc.parallel_loop(0, chunk, 1)
                def body(r):
                    for j in range(vecs_per_row):
                        xbuf[r, pl.ds(j * VEC, VEC)] = (
                            xbuf[r, pl.ds(j * VEC, VEC)]
                            + posbuf[r, pl.ds(j * VEC, VEC)])

                pltpu.sync_copy(xbuf, out_hbm.at[b, pl.ds(row0, chunk)])

    return run(x, pos_embedding, sp)


# speedup vs baseline: 3.3511x; 1.6629x over previous
import functools

import jax
import jax.numpy as jnp
from jax import lax
from jax.experimental import pallas as pl
from jax.experimental.pallas import tpu as pltpu
from jax.experimental.pallas import tpu_sc as plsc

NUM_CORES = 2
NUM_SUBCORES = 16
NUM_WORKERS = NUM_CORES * NUM_SUBCORES
VEC = 16
NBUF = 3


def kernel(x, pos_embedding, start_pos):
    batch, seq_len, d_model = x.shape
    rows_per_worker = seq_len // NUM_WORKERS
    chunk = min(16, rows_per_worker)
    n_chunks = rows_per_worker // chunk
    vecs_per_row = d_model // VEC
    n_vecs = chunk * vecs_per_row
    row_shift = vecs_per_row.bit_length() - 1

    sp = jnp.full((16,), start_pos, dtype=jnp.int32)

    mesh = plsc.VectorSubcoreMesh(
        core_axis_name="c", subcore_axis_name="s",
        num_cores=NUM_CORES, num_subcores=NUM_SUBCORES)

    @functools.partial(
        pl.kernel,
        out_type=jax.ShapeDtypeStruct((batch, seq_len, d_model),
                                      jnp.float32),
        mesh=mesh,
        scratch_types=[
            pltpu.VMEM((16,), jnp.int32),
            [pltpu.VMEM((chunk, d_model), jnp.float32)] * 2,
            [pltpu.VMEM((chunk, d_model), jnp.float32)] * NBUF,
            [pltpu.SemaphoreType.DMA] * 2,
            [pltpu.SemaphoreType.DMA] * NBUF,
            [pltpu.SemaphoreType.DMA] * NBUF,
        ],
        compiler_params=pltpu.CompilerParams(use_tc_tiling_on_sc=True),
    )
    def run(x_hbm, pos_hbm, sp_hbm, out_hbm, sp_vmem, posbufs, xbufs,
            pos_sems, ld_sems, st_sems):
        cid = lax.axis_index("c")
        sid = lax.axis_index("s")
        wid = sid * NUM_CORES + cid
        pltpu.sync_copy(sp_hbm, sp_vmem)
        s0 = sp_vmem[...][0]
        base = wid * rows_per_worker

        def rows(c):
            return pl.multiple_of(base + c * chunk, chunk)

        def start_pos_load(c):
            prow = pl.multiple_of(s0 + rows(c), 8)
            return pltpu.async_copy(
                pos_hbm.at[pl.ds(prow, chunk)], posbufs[c % 2],
                pos_sems[c % 2])

        def start_x_load(t):
            c, b = divmod(t, batch)
            return pltpu.async_copy(
                x_hbm.at[b, pl.ds(rows(c), chunk)], xbufs[t % NBUF],
                ld_sems[t % NBUF])

        n_steps = n_chunks * batch
        pos_d = {0: start_pos_load(0)}
        ld_d = {0: start_x_load(0)}
        st_d = {}
        for t in range(n_steps):
            c, b = divmod(t, batch)
            if b == 0:
                if c + 1 < n_chunks:
                    pos_d[c + 1] = start_pos_load(c + 1)
                pos_d[c].wait()
            if t + 1 < n_steps:
                if t + 1 - NBUF in st_d:
                    st_d[t + 1 - NBUF].wait()
                ld_d[t + 1] = start_x_load(t + 1)
            ld_d[t].wait()
            xbuf, posbuf = xbufs[t % NBUF], posbufs[c % 2]

            @plsc.parallel_loop(0, n_vecs, 1, unroll=4)
            def body(i):
                r = lax.shift_right_logical(i, row_shift)
                col = lax.mul(lax.rem(i, vecs_per_row), VEC)
                xbuf[r, pl.ds(col, VEC)] = (
                    xbuf[r, pl.ds(col, VEC)] + posbuf[r, pl.ds(col, VEC)])

            st_d[t] = pltpu.async_copy(
                xbufs[t % NBUF], out_hbm.at[b, pl.ds(rows(c), chunk)],
                st_sems[t % NBUF])
        for t in range(max(0, n_steps - NBUF), n_steps):
            st_d[t].wait()

    return run(x, pos_embedding, sp)
